# baseline (speedup 1.0000x reference)
import jax
import jax.numpy as jnp
from jax import lax
from jax.experimental import pallas as pl
from jax.experimental.pallas import tpu as pltpu
from jax.experimental.pallas import tpu_sc as plsc

PATCH = 32
DIM = 256
N = 256
GRIDW = 16
INV_TEMP = 10.0
L = 16
BIG = 1 << 30


def _scores_body(i0_ref, i1_ref, w_ref, out_ref):
    w = w_ref[...]

    def features(img):
        x4 = img.reshape(GRIDW, PATCH, GRIDW, PATCH)
        w3 = w.reshape(PATCH, PATCH, DIM)
        c = lax.dot_general(
            x4, w3, (((3,), (1,)), ((1,), (0,))),
            preferred_element_type=jnp.float32,
        )
        f = jnp.sum(c, axis=0).reshape(N, DIM)
        return f / (jnp.sqrt(jnp.sum(f * f, axis=1, keepdims=True)) + 1e-6)

    f0 = features(i0_ref[0])
    f1 = features(i1_ref[0])
    sim = lax.dot_general(
        f0, f1, (((1,), (1,)), ((), ())), preferred_element_type=jnp.float32
    ) * INV_TEMP
    rmax = jnp.max(sim, axis=1, keepdims=True)
    e_r = jnp.exp(sim - rmax)
    sm_r = e_r / jnp.sum(e_r, axis=1, keepdims=True)
    cmax = jnp.max(sim, axis=0, keepdims=True)
    e_c = jnp.exp(sim - cmax)
    sm_c = e_c / jnp.sum(e_c, axis=0, keepdims=True)
    out_ref[0] = sm_r * sm_c


def _scores_tc(image0, image1, W_proj):
    B, H, Wd = image0.shape
    return pl.pallas_call(
        _scores_body,
        grid=(B,),
        in_specs=[
            pl.BlockSpec((1, H, Wd), lambda b: (b, 0, 0)),
            pl.BlockSpec((1, H, Wd), lambda b: (b, 0, 0)),
            pl.BlockSpec((PATCH * PATCH, DIM), lambda b: (0, 0)),
        ],
        out_specs=pl.BlockSpec((1, N, N), lambda b: (b, 0, 0)),
        out_shape=jax.ShapeDtypeStruct((B, N, N), jnp.float32),
    )(image0, image1, W_proj)


def _sc_match_body(scores_hbm, out_hbm, rows_v, pm_v, pa_v, part_m, part_a,
                   cm_v, ca_v, jbuf, obuf, mbuf, ibuf):
    b = lax.axis_index("c")
    ww = lax.axis_index("s")
    rbase = ww * L

    pltpu.sync_copy(scores_hbm.at[b, pl.ds(rbase, L)], rows_v)

    lane = lax.iota(jnp.int32, L)
    zf = jnp.zeros((L,), jnp.float32)
    zi = jnp.zeros((L,), jnp.int32)

    col_m = [zf - 2.0 for _ in range(16)]
    col_a = [zi for _ in range(16)]
    for r in range(L):
        row_m = zf - 2.0
        row_i = zi
        for c in range(16):
            v = rows_v[r, pl.ds(c * L, L)]
            rc = v > row_m
            row_m = jnp.where(rc, v, row_m)
            row_i = jnp.where(rc, c * L + lane, row_i)
            cc = v > col_m[c]
            col_m[c] = jnp.where(cc, v, col_m[c])
            col_a[c] = jnp.where(
                cc, jnp.full((L,), rbase + r, jnp.int32), col_a[c])
        mbuf[pl.ds(r * L, L)] = row_m
        ibuf[pl.ds(r * L, L)] = row_i

    conf_acc = zf - 2.0
    i2j_acc = zi + BIG
    for l in range(L):
        vm = plsc.load_gather(mbuf, [lane * L + l])
        vi = plsc.load_gather(ibuf, [lane * L + l])
        take = jnp.logical_or(
            vm > conf_acc,
            jnp.logical_and(vm == conf_acc, vi < i2j_acc))
        conf_acc = jnp.where(take, vm, conf_acc)
        i2j_acc = jnp.where(take, vi, i2j_acc)

    for c in range(16):
        cm_v[pl.ds(c * L, L)] = col_m[c]
        ca_v[pl.ds(c * L, L)] = col_a[c]
    pltpu.sync_copy(cm_v, part_m.at[ww])
    pltpu.sync_copy(ca_v, part_a.at[ww])
    plsc.subcore_barrier()
    pltpu.sync_copy(part_m, pm_v)
    pltpu.sync_copy(part_a, pa_v)

    for c in range(16):
        gm = zf - 2.0
        ga = zi
        for w2 in range(16):
            pm = pm_v[w2, pl.ds(c * L, L)]
            pa = pa_v[w2, pl.ds(c * L, L)]
            cc = pm > gm
            gm = jnp.where(cc, pm, gm)
            ga = jnp.where(cc, pa, ga)
        jbuf[pl.ds(c * L, L)] = ga

    back = plsc.load_gather(jbuf, [i2j_acc])
    il = rbase + lane
    mutual = back == il
    valid = jnp.logical_and(mutual, conf_acc > 1e-6)
    wt = conf_acc * valid.astype(jnp.float32)

    half = jnp.float32(PATCH // 2)
    xl = (il % GRIDW).astype(jnp.float32) * PATCH + half
    yl = (il // GRIDW).astype(jnp.float32) * PATCH + half
    xr = (i2j_acc % GRIDW).astype(jnp.float32) * PATCH + half
    yr = (i2j_acc // GRIDW).astype(jnp.float32) * PATCH + half

    idx4 = lane * 4
    plsc.store_scatter(obuf, [idx4], xl * wt)
    plsc.store_scatter(obuf, [idx4 + 1], yl * wt)
    plsc.store_scatter(obuf, [idx4 + 2], xr * wt)
    plsc.store_scatter(obuf, [idx4 + 3], yr * wt)
    pltpu.sync_copy(obuf, out_hbm.at[pl.ds((b * N + rbase) * 4, L * 4)])


def _match_sc(scores):
    B = scores.shape[0]
    mesh = plsc.VectorSubcoreMesh(core_axis_name="c", subcore_axis_name="s")
    kern = pl.kernel(
        _sc_match_body,
        mesh=mesh,
        compiler_params=pltpu.CompilerParams(needs_layout_passes=False),
        out_type=jax.ShapeDtypeStruct((B * N * 4,), jnp.float32),
        scratch_types=[
            pltpu.VMEM((L, N), jnp.float32),
            pltpu.VMEM((L, N), jnp.float32),
            pltpu.VMEM((L, N), jnp.int32),
            pltpu.VMEM_SHARED((L, N), jnp.float32),
            pltpu.VMEM_SHARED((L, N), jnp.int32),
            pltpu.VMEM((N,), jnp.float32),
            pltpu.VMEM((N,), jnp.int32),
            pltpu.VMEM((N,), jnp.int32),
            pltpu.VMEM((L * 4,), jnp.float32),
            pltpu.VMEM((L * L,), jnp.float32),
            pltpu.VMEM((L * L,), jnp.int32),
        ],
    )
    return kern(scores)


def kernel(image0, image1, W_proj):
    scores = _scores_tc(image0, image1, W_proj)
    out = _match_sc(scores)
    return out.reshape(image0.shape[0], N, 4)
